# Initial kernel scaffold; baseline (speedup 1.0000x reference)
import jax
import jax.numpy as jnp
from jax.experimental import pallas as pl


def kernel(feats, edge_index, node2graph, W0, b0, W1, b1, W2, b2, Ww, bw, Wp1, bp1, bn_gamma, bn_beta, bn_mean, bn_var, Wp2, bp2):
    raise NotImplementedError("write your pallas kernel here")


# speedup vs baseline: 1.8196x; 1.8196x over previous
import functools

import jax
import jax.numpy as jnp
from jax import lax
from jax.experimental import pallas as pl
from jax.experimental.pallas import tpu as pltpu
from jax.experimental.pallas import tpu_sc as plsc

N = 10000
E = 320000
G = 64

CHUNK = 128
NCHUNK = 2560
EPAD = NCHUNK * CHUNK
NTILE = 32
CH_PER_TILE = NCHUNK // NTILE
NP = 10240
ROWS_PER_TILE = NP // 16

BN = 1000
NB = N // BN


def _sc_agg(table, src2d, dst2d, zeros_blk):
    mesh = plsc.VectorSubcoreMesh(core_axis_name="c", subcore_axis_name="s",
                                  num_cores=2, num_subcores=16)

    @functools.partial(
        pl.kernel,
        mesh=mesh,
        out_type=jax.ShapeDtypeStruct((2, NP, 128), jnp.float32),
        scratch_types=[
            pltpu.VMEM((CH_PER_TILE, CHUNK), jnp.int32),
            pltpu.VMEM((CH_PER_TILE, CHUNK), jnp.int32),
            pltpu.VMEM((CHUNK, 128), jnp.float32),
            pltpu.VMEM_SHARED((NP, 128), jnp.float32),
            pltpu.SemaphoreType.DMA,
        ],
    )
    def k(table_hbm, src_hbm, dst_hbm, zero_hbm, out_hbm,
          src_v, dst_v, rows_v, acc_sh, sem):
        cid = lax.axis_index("c")
        sid = lax.axis_index("s")
        wid = sid * 2 + cid
        rbase = sid * ROWS_PER_TILE
        pltpu.sync_copy(zero_hbm, acc_sh.at[pl.ds(rbase, ROWS_PER_TILE)])
        cbase = wid * CH_PER_TILE
        pltpu.sync_copy(src_hbm.at[pl.ds(cbase, CH_PER_TILE)], src_v)
        pltpu.sync_copy(dst_hbm.at[pl.ds(cbase, CH_PER_TILE)], dst_v)
        plsc.subcore_barrier()

        def body(j, carry):
            pltpu.async_copy(table_hbm.at[src_v.at[j]], rows_v, sem).wait()
            pltpu.sync_copy(rows_v, acc_sh.at[dst_v.at[j]], add=True)
            return carry

        lax.fori_loop(0, CH_PER_TILE, body, 0)
        plsc.subcore_barrier()

        @pl.when(cid == 0)
        def _():
            pltpu.sync_copy(acc_sh.at[pl.ds(rbase, ROWS_PER_TILE)],
                            out_hbm.at[0, pl.ds(rbase, ROWS_PER_TILE)])

        @pl.when(cid == 1)
        def _():
            pltpu.sync_copy(acc_sh.at[pl.ds(rbase, ROWS_PER_TILE)],
                            out_hbm.at[1, pl.ds(rbase, ROWS_PER_TILE)])

    return k(table, src2d, dst2d, zeros_blk)


def _wcols():
    return [pl.BlockSpec((128, 128), lambda i: (0, 0))]


def _tc_m0(feats, W0):
    def body(x_ref, w_ref, o_ref):
        o_ref[...] = jnp.dot(x_ref[...], w_ref[...],
                             preferred_element_type=jnp.float32)

    return pl.pallas_call(
        body,
        grid=(NB,),
        in_specs=[pl.BlockSpec((BN, 128), lambda i: (i, 0)),
                  pl.BlockSpec((128, 128), lambda i: (0, 0))],
        out_specs=pl.BlockSpec((BN, 128), lambda i: (i, 0)),
        out_shape=jax.ShapeDtypeStruct((N, 128), jnp.float32),
    )(feats, W0)


def _tc_layer1(pA, b0, W1):
    def body(p_ref, b_ref, wa_ref, wb_ref, oa_ref, ob_ref):
        h = jnp.maximum(p_ref[0] + p_ref[1] + b_ref[...], 0.0)
        oa_ref[...] = jnp.dot(h, wa_ref[...],
                              preferred_element_type=jnp.float32)
        ob_ref[...] = jnp.dot(h, wb_ref[...],
                              preferred_element_type=jnp.float32)

    return pl.pallas_call(
        body,
        grid=(NB,),
        in_specs=[pl.BlockSpec((2, BN, 128), lambda i: (0, i, 0)),
                  pl.BlockSpec((1, 128), lambda i: (0, 0)),
                  pl.BlockSpec((128, 128), lambda i: (0, 0)),
                  pl.BlockSpec((128, 128), lambda i: (0, 0))],
        out_specs=[pl.BlockSpec((BN, 128), lambda i: (i, 0))] * 2,
        out_shape=[jax.ShapeDtypeStruct((N, 128), jnp.float32)] * 2,
    )(pA, b0, W1[:, :128], W1[:, 128:])


def _tc_layer2(pBlo, pBhi, b1, W2):
    def body(plo_ref, phi_ref, b_ref, w0_ref, w1_ref, w2_ref, w3_ref,
             *o_refs):
        h = jnp.concatenate([plo_ref[0] + plo_ref[1],
                             phi_ref[0] + phi_ref[1]], axis=1)
        h = jnp.maximum(h + b_ref[...], 0.0)
        for o_ref, w_ref in zip(o_refs, (w0_ref, w1_ref, w2_ref, w3_ref)):
            o_ref[...] = jnp.dot(h, w_ref[...],
                                 preferred_element_type=jnp.float32)

    return pl.pallas_call(
        body,
        grid=(NB,),
        in_specs=[pl.BlockSpec((2, BN, 128), lambda i: (0, i, 0)),
                  pl.BlockSpec((2, BN, 128), lambda i: (0, i, 0)),
                  pl.BlockSpec((1, 256), lambda i: (0, 0))] +
                 [pl.BlockSpec((256, 128), lambda i: (0, 0))] * 4,
        out_specs=[pl.BlockSpec((BN, 128), lambda i: (i, 0))] * 4,
        out_shape=[jax.ShapeDtypeStruct((N, 128), jnp.float32)] * 4,
    )(pBlo, pBhi, b1, *(W2[:, 128 * j: 128 * (j + 1)] for j in range(4)))


def _tc_fused(pCs, n2g3d, n2g3c, b2, Ww, bw2, Wp1s, Wp1m, bp1,
              bng, bnb, bnm, bnv, Wp2, bp2):
    def body(p0_ref, p1_ref, p2_ref, p3_ref, n2g_ref, n2gc_ref, b2_ref,
             ww_ref, bw_ref, wp1s_ref, wp1m_ref, bp1_ref, bng_ref, bnb_ref,
             bnm_ref, bnv_ref, wp2_ref, bp2_ref, o_ref, hsum_ref, hmax_ref):
        i = pl.program_id(0)

        @pl.when(i == 0)
        def _():
            hsum_ref[...] = jnp.zeros_like(hsum_ref)
            hmax_ref[...] = jnp.full_like(hmax_ref, -jnp.inf)

        h2 = jnp.concatenate(
            [r[0] + r[1] for r in (p0_ref, p1_ref, p2_ref, p3_ref)], axis=1)
        h2 = jnp.maximum(h2 + b2_ref[...], 0.0)

        wlog = jnp.dot(h2, ww_ref[...],
                       preferred_element_type=jnp.float32)
        wcol = jax.nn.sigmoid(wlog + bw_ref[0, 0])

        n2gr = n2g_ref[0]
        n2gc = n2gc_ref[0]
        gids = lax.broadcasted_iota(jnp.int32, (G, BN), 0)
        maskf = (gids == n2gr).astype(jnp.float32)
        hsum_ref[...] += jnp.dot(maskf, h2 * wcol,
                                 preferred_element_type=jnp.float32,
                                 precision=lax.Precision.HIGHEST)

        gmin = jnp.min(n2gr)
        gmax = jnp.max(n2gr)

        def gbody(g, carry):
            m = n2gc == g
            cur = jnp.max(jnp.where(m, h2, -jnp.inf), axis=0,
                          keepdims=True)
            hmax_ref[pl.ds(g, 1), :] = jnp.maximum(hmax_ref[pl.ds(g, 1), :],
                                                   cur)
            return carry

        lax.fori_loop(gmin, gmax + 1, gbody, 0)

        @pl.when(i == NB - 1)
        def _():
            x = jnp.dot(hsum_ref[...], wp1s_ref[...],
                        preferred_element_type=jnp.float32)
            x = x + jnp.dot(hmax_ref[...], wp1m_ref[...],
                            preferred_element_type=jnp.float32)
            x = jnp.maximum(x + bp1_ref[...], 0.0)
            x = (x - bnm_ref[...]) / jnp.sqrt(bnv_ref[...] + 1e-5)
            x = x * bng_ref[...] + bnb_ref[...]
            o_ref[...] = jnp.dot(x, wp2_ref[...],
                                 preferred_element_type=jnp.float32) \
                + bp2_ref[0, 0]

    return pl.pallas_call(
        body,
        grid=(NB,),
        in_specs=[pl.BlockSpec((2, BN, 128), lambda i: (0, i, 0))] * 4 + [
            pl.BlockSpec((1, 1, BN), lambda i: (i, 0, 0)),
            pl.BlockSpec((1, BN, 1), lambda i: (i, 0, 0)),
            pl.BlockSpec((1, 512), lambda i: (0, 0)),
            pl.BlockSpec((512, 1), lambda i: (0, 0)),
            pl.BlockSpec((1, 1), lambda i: (0, 0)),
            pl.BlockSpec((512, 128), lambda i: (0, 0)),
            pl.BlockSpec((512, 128), lambda i: (0, 0)),
            pl.BlockSpec((1, 128), lambda i: (0, 0)),
            pl.BlockSpec((1, 128), lambda i: (0, 0)),
            pl.BlockSpec((1, 128), lambda i: (0, 0)),
            pl.BlockSpec((1, 128), lambda i: (0, 0)),
            pl.BlockSpec((1, 128), lambda i: (0, 0)),
            pl.BlockSpec((128, 1), lambda i: (0, 0)),
            pl.BlockSpec((1, 1), lambda i: (0, 0)),
        ],
        out_specs=pl.BlockSpec((G, 1), lambda i: (0, 0)),
        out_shape=jax.ShapeDtypeStruct((G, 1), jnp.float32),
        scratch_shapes=[
            pltpu.VMEM((G, 512), jnp.float32),
            pltpu.VMEM((G, 512), jnp.float32),
        ],
    )(*pCs, n2g3d, n2g3c, b2, Ww, bw2, Wp1s, Wp1m, bp1,
      bng, bnb, bnm, bnv, Wp2, bp2)


def kernel(feats, edge_index, node2graph, W0, b0, W1, b1, W2, b2, Ww, bw,
           Wp1, bp1, bn_gamma, bn_beta, bn_mean, bn_var, Wp2, bp2):
    pad = EPAD - E
    src2d = jnp.concatenate(
        [edge_index[0], jnp.zeros((pad,), jnp.int32)]).reshape(NCHUNK, CHUNK)
    dst2d = jnp.concatenate(
        [edge_index[1], jnp.full((pad,), N, jnp.int32)]).reshape(NCHUNK, CHUNK)
    zeros_blk = jnp.zeros((ROWS_PER_TILE, 128), jnp.float32)
    n2g3d = node2graph.reshape(NB, 1, BN)
    n2g3c = node2graph.reshape(NB, BN, 1)

    m0 = _tc_m0(feats, W0)
    pA = _sc_agg(m0, src2d, dst2d, zeros_blk)
    m1lo, m1hi = _tc_layer1(pA, b0.reshape(1, -1), W1)
    pBlo = _sc_agg(m1lo, src2d, dst2d, zeros_blk)
    pBhi = _sc_agg(m1hi, src2d, dst2d, zeros_blk)
    m2 = _tc_layer2(pBlo, pBhi, b1.reshape(1, -1), W2)
    pCs = [_sc_agg(m, src2d, dst2d, zeros_blk) for m in m2]

    return _tc_fused(
        pCs, n2g3d, n2g3c, b2.reshape(1, -1),
        Ww, bw.reshape(1, 1),
        Wp1[:512], Wp1[512:], bp1.reshape(1, -1),
        bn_gamma.reshape(1, -1), bn_beta.reshape(1, -1),
        bn_mean.reshape(1, -1), bn_var.reshape(1, -1),
        Wp2, bp2.reshape(1, 1))


# speedup vs baseline: 2.0313x; 1.1164x over previous
import functools

import jax
import jax.numpy as jnp
from jax import lax
from jax.experimental import pallas as pl
from jax.experimental.pallas import tpu as pltpu
from jax.experimental.pallas import tpu_sc as plsc

N = 10000
E = 320000
G = 64

CHUNK = 128
NCHUNK = 2560
EPAD = NCHUNK * CHUNK
NTILE = 32
CH_PER_TILE = NCHUNK // NTILE
NP = 10240
ROWS_PER_TILE = NP // 16

BN = 1000
NB = N // BN


def _sc_agg(table, src2d, dst2d, zeros_blk):
    mesh = plsc.VectorSubcoreMesh(core_axis_name="c", subcore_axis_name="s",
                                  num_cores=2, num_subcores=16)

    @functools.partial(
        pl.kernel,
        mesh=mesh,
        out_type=jax.ShapeDtypeStruct((2, NP, 128), jnp.float32),
        scratch_types=[
            pltpu.VMEM((CH_PER_TILE // 2, CHUNK), jnp.int32),
            pltpu.VMEM((CH_PER_TILE // 2, CHUNK), jnp.int32),
            pltpu.VMEM((CHUNK, 128), jnp.float32),
            pltpu.VMEM((CHUNK, 128), jnp.float32),
            pltpu.VMEM_SHARED((NP, 128), jnp.float32),
        ] + [pltpu.SemaphoreType.DMA] * 2,
    )
    def k(table_hbm, src_hbm, dst_hbm, zero_hbm, out_hbm,
          src_v, dst_v, r0, r1, acc_sh,
          g0, g1):
        cid = lax.axis_index("c")
        sid = lax.axis_index("s")
        wid = sid * 2 + cid
        rbase = sid * ROWS_PER_TILE
        pltpu.sync_copy(zero_hbm, acc_sh.at[pl.ds(rbase, ROWS_PER_TILE)])
        cbase = wid * CH_PER_TILE
        plsc.subcore_barrier()

        def gat(j, buf, sem):
            pltpu.async_copy(table_hbm.at[src_v.at[j]], buf, sem)

        def gat_wait(j, buf, sem):
            pltpu.make_async_copy(table_hbm.at[src_v.at[j]], buf, sem).wait()

        HP = CH_PER_TILE // 2
        for p in range(2):
            pbase = cbase + p * HP
            pltpu.sync_copy(src_hbm.at[pl.ds(pbase, HP)], src_v)
            pltpu.sync_copy(dst_hbm.at[pl.ds(pbase, HP)], dst_v)

            for b, (rb, gb) in enumerate(((r0, g0), (r1, g1))):
                gat(b, rb, gb)

            def body(s, carry):
                for b, (rb, gb) in enumerate(((r0, g0), (r1, g1))):
                    j = 2 * s + b
                    gat_wait(j, rb, gb)
                    pltpu.sync_copy(rb, acc_sh.at[dst_v.at[j]], add=True)

                    @pl.when(j + 2 < HP)
                    def _():
                        gat(j + 2, rb, gb)

                return carry

            lax.fori_loop(0, HP // 2, body, 0)

        plsc.subcore_barrier()

        @pl.when(cid == 0)
        def _():
            pltpu.sync_copy(acc_sh.at[pl.ds(rbase, ROWS_PER_TILE)],
                            out_hbm.at[0, pl.ds(rbase, ROWS_PER_TILE)])

        @pl.when(cid == 1)
        def _():
            pltpu.sync_copy(acc_sh.at[pl.ds(rbase, ROWS_PER_TILE)],
                            out_hbm.at[1, pl.ds(rbase, ROWS_PER_TILE)])

    return k(table, src2d, dst2d, zeros_blk)


def _tc_m0(feats, W0):
    def body(x_ref, w_ref, o_ref):
        o_ref[...] = jnp.dot(x_ref[...], w_ref[...],
                             preferred_element_type=jnp.float32)

    return pl.pallas_call(
        body,
        grid=(NB,),
        in_specs=[pl.BlockSpec((BN, 128), lambda i: (i, 0)),
                  pl.BlockSpec((128, 128), lambda i: (0, 0))],
        out_specs=pl.BlockSpec((BN, 128), lambda i: (i, 0)),
        out_shape=jax.ShapeDtypeStruct((N, 128), jnp.float32),
    )(feats, W0)


def _tc_layer1(pA, b0, W1):
    def body(p_ref, b_ref, wa_ref, wb_ref, oa_ref, ob_ref):
        h = jnp.maximum(p_ref[0] + p_ref[1] + b_ref[...], 0.0)
        oa_ref[...] = jnp.dot(h, wa_ref[...],
                              preferred_element_type=jnp.float32)
        ob_ref[...] = jnp.dot(h, wb_ref[...],
                              preferred_element_type=jnp.float32)

    return pl.pallas_call(
        body,
        grid=(NB,),
        in_specs=[pl.BlockSpec((2, BN, 128), lambda i: (0, i, 0)),
                  pl.BlockSpec((1, 128), lambda i: (0, 0)),
                  pl.BlockSpec((128, 128), lambda i: (0, 0)),
                  pl.BlockSpec((128, 128), lambda i: (0, 0))],
        out_specs=[pl.BlockSpec((BN, 128), lambda i: (i, 0))] * 2,
        out_shape=[jax.ShapeDtypeStruct((N, 128), jnp.float32)] * 2,
    )(pA, b0, W1[:, :128], W1[:, 128:])


def _tc_layer2(pBlo, pBhi, b1, W2):
    def body(plo_ref, phi_ref, b_ref, w0_ref, w1_ref, w2_ref, w3_ref,
             *o_refs):
        h = jnp.concatenate([plo_ref[0] + plo_ref[1],
                             phi_ref[0] + phi_ref[1]], axis=1)
        h = jnp.maximum(h + b_ref[...], 0.0)
        for o_ref, w_ref in zip(o_refs, (w0_ref, w1_ref, w2_ref, w3_ref)):
            o_ref[...] = jnp.dot(h, w_ref[...],
                                 preferred_element_type=jnp.float32)

    return pl.pallas_call(
        body,
        grid=(NB,),
        in_specs=[pl.BlockSpec((2, BN, 128), lambda i: (0, i, 0)),
                  pl.BlockSpec((2, BN, 128), lambda i: (0, i, 0)),
                  pl.BlockSpec((1, 256), lambda i: (0, 0))] +
                 [pl.BlockSpec((256, 128), lambda i: (0, 0))] * 4,
        out_specs=[pl.BlockSpec((BN, 128), lambda i: (i, 0))] * 4,
        out_shape=[jax.ShapeDtypeStruct((N, 128), jnp.float32)] * 4,
    )(pBlo, pBhi, b1, *(W2[:, 128 * j: 128 * (j + 1)] for j in range(4)))


def _tc_fused(pCs, n2g3d, n2g3c, b2, Ww, bw2, Wp1s, Wp1m, bp1,
              bng, bnb, bnm, bnv, Wp2, bp2):
    def body(p0_ref, p1_ref, p2_ref, p3_ref, n2g_ref, n2gc_ref, b2_ref,
             ww_ref, bw_ref, wp1s_ref, wp1m_ref, bp1_ref, bng_ref, bnb_ref,
             bnm_ref, bnv_ref, wp2_ref, bp2_ref, o_ref, hsum_ref, hmax_ref):
        i = pl.program_id(0)

        @pl.when(i == 0)
        def _():
            hsum_ref[...] = jnp.zeros_like(hsum_ref)
            hmax_ref[...] = jnp.full_like(hmax_ref, -jnp.inf)

        h2 = jnp.concatenate(
            [r[0] + r[1] for r in (p0_ref, p1_ref, p2_ref, p3_ref)], axis=1)
        h2 = jnp.maximum(h2 + b2_ref[...], 0.0)

        wlog = jnp.dot(h2, ww_ref[...],
                       preferred_element_type=jnp.float32)
        wcol = jax.nn.sigmoid(wlog + bw_ref[0, 0])

        n2gr = n2g_ref[0]
        n2gc = n2gc_ref[0]
        gids = lax.broadcasted_iota(jnp.int32, (G, BN), 0)
        maskf = (gids == n2gr).astype(jnp.float32)
        hsum_ref[...] += jnp.dot(maskf, h2 * wcol,
                                 preferred_element_type=jnp.float32,
                                 precision=lax.Precision.HIGHEST)

        gmin = jnp.min(n2gr)
        gmax = jnp.max(n2gr)

        def gbody(g, carry):
            m = n2gc == g
            cur = jnp.max(jnp.where(m, h2, -jnp.inf), axis=0,
                          keepdims=True)
            hmax_ref[pl.ds(g, 1), :] = jnp.maximum(hmax_ref[pl.ds(g, 1), :],
                                                   cur)
            return carry

        lax.fori_loop(gmin, gmax + 1, gbody, 0)

        @pl.when(i == NB - 1)
        def _():
            x = jnp.dot(hsum_ref[...], wp1s_ref[...],
                        preferred_element_type=jnp.float32)
            x = x + jnp.dot(hmax_ref[...], wp1m_ref[...],
                            preferred_element_type=jnp.float32)
            x = jnp.maximum(x + bp1_ref[...], 0.0)
            x = (x - bnm_ref[...]) / jnp.sqrt(bnv_ref[...] + 1e-5)
            x = x * bng_ref[...] + bnb_ref[...]
            o_ref[...] = jnp.dot(x, wp2_ref[...],
                                 preferred_element_type=jnp.float32) \
                + bp2_ref[0, 0]

    return pl.pallas_call(
        body,
        grid=(NB,),
        in_specs=[pl.BlockSpec((2, BN, 128), lambda i: (0, i, 0))] * 4 + [
            pl.BlockSpec((1, 1, BN), lambda i: (i, 0, 0)),
            pl.BlockSpec((1, BN, 1), lambda i: (i, 0, 0)),
            pl.BlockSpec((1, 512), lambda i: (0, 0)),
            pl.BlockSpec((512, 1), lambda i: (0, 0)),
            pl.BlockSpec((1, 1), lambda i: (0, 0)),
            pl.BlockSpec((512, 128), lambda i: (0, 0)),
            pl.BlockSpec((512, 128), lambda i: (0, 0)),
            pl.BlockSpec((1, 128), lambda i: (0, 0)),
            pl.BlockSpec((1, 128), lambda i: (0, 0)),
            pl.BlockSpec((1, 128), lambda i: (0, 0)),
            pl.BlockSpec((1, 128), lambda i: (0, 0)),
            pl.BlockSpec((1, 128), lambda i: (0, 0)),
            pl.BlockSpec((128, 1), lambda i: (0, 0)),
            pl.BlockSpec((1, 1), lambda i: (0, 0)),
        ],
        out_specs=pl.BlockSpec((G, 1), lambda i: (0, 0)),
        out_shape=jax.ShapeDtypeStruct((G, 1), jnp.float32),
        scratch_shapes=[
            pltpu.VMEM((G, 512), jnp.float32),
            pltpu.VMEM((G, 512), jnp.float32),
        ],
    )(*pCs, n2g3d, n2g3c, b2, Ww, bw2, Wp1s, Wp1m, bp1,
      bng, bnb, bnm, bnv, Wp2, bp2)


def kernel(feats, edge_index, node2graph, W0, b0, W1, b1, W2, b2, Ww, bw,
           Wp1, bp1, bn_gamma, bn_beta, bn_mean, bn_var, Wp2, bp2):
    pad = EPAD - E
    src2d = jnp.concatenate(
        [edge_index[0], jnp.zeros((pad,), jnp.int32)]).reshape(NCHUNK, CHUNK)
    dst2d = jnp.concatenate(
        [edge_index[1], jnp.full((pad,), N, jnp.int32)]).reshape(NCHUNK, CHUNK)
    zeros_blk = jnp.zeros((ROWS_PER_TILE, 128), jnp.float32)
    n2g3d = node2graph.reshape(NB, 1, BN)
    n2g3c = node2graph.reshape(NB, BN, 1)

    m0 = _tc_m0(feats, W0)
    pA = _sc_agg(m0, src2d, dst2d, zeros_blk)
    m1lo, m1hi = _tc_layer1(pA, b0.reshape(1, -1), W1)
    pBlo = _sc_agg(m1lo, src2d, dst2d, zeros_blk)
    pBhi = _sc_agg(m1hi, src2d, dst2d, zeros_blk)
    m2 = _tc_layer2(pBlo, pBhi, b1.reshape(1, -1), W2)
    pCs = [_sc_agg(m, src2d, dst2d, zeros_blk) for m in m2]

    return _tc_fused(
        pCs, n2g3d, n2g3c, b2.reshape(1, -1),
        Ww, bw.reshape(1, 1),
        Wp1[:512], Wp1[512:], bp1.reshape(1, -1),
        bn_gamma.reshape(1, -1), bn_beta.reshape(1, -1),
        bn_mean.reshape(1, -1), bn_var.reshape(1, -1),
        Wp2, bp2.reshape(1, 1))
